# baseline (speedup 1.0000x reference)
import functools

import jax
import jax.numpy as jnp
from jax.experimental import pallas as pl
from jax.experimental.pallas import tpu as pltpu


def _mlp_kernel(x_ref, w1_ref, b1_ref, w2_ref, b2_ref, w3_ref, b3_ref, o_ref):
    x = x_ref[...]
    h1 = jnp.dot(x, w1_ref[...], preferred_element_type=jnp.float32)
    h1 = jnp.maximum(h1 + b1_ref[...], 0.0)
    h2 = jnp.dot(h1, w2_ref[...], preferred_element_type=jnp.float32)
    h2 = jnp.maximum(h2 + b2_ref[...], 0.0)
    out = jnp.dot(h2, w3_ref[...], preferred_element_type=jnp.float32)
    o_ref[...] = out + b3_ref[...]


def _expand(w, p=4):
    return jnp.kron(jnp.eye(p, dtype=w.dtype), w.T)


@functools.partial(jax.jit, static_argnames=("rb",))
def _run(x, w1, b1, w2, b2, w3, b3, *, rb=2048):
    B, F = x.shape
    rows = B // 4
    xr = x.reshape(rows, 16)
    w1t = _expand(w1)
    w2t = _expand(w2)
    w3t = _expand(w3)
    b1t = jnp.tile(b1[:, 0], 4)[None, :]
    b2t = jnp.tile(b2[:, 0], 4)[None, :]
    b3t = jnp.tile(b3[:, 0], 4)[None, :]
    n_steps = rows // rb
    const = lambda a: pl.BlockSpec(a.shape, lambda i: (0, 0))
    out = pl.pallas_call(
        _mlp_kernel,
        out_shape=jax.ShapeDtypeStruct((rows, 12), jnp.float32),
        grid=(n_steps,),
        in_specs=[
            pl.BlockSpec((rb, 16), lambda i: (i, 0)),
            const(w1t), const(b1t),
            const(w2t), const(b2t),
            const(w3t), const(b3t),
        ],
        out_specs=pl.BlockSpec((rb, 12), lambda i: (i, 0)),
        compiler_params=pltpu.CompilerParams(
            dimension_semantics=("parallel",),
        ),
    )(xr, w1t, b1t, w2t, b2t, w3t, b3t)
    return out.reshape(B, 3)


def kernel(x, w1, b1, w2, b2, w3, b3):
    return _run(x, w1, b1, w2, b2, w3, b3)
